# Initial kernel scaffold; baseline (speedup 1.0000x reference)
import jax
import jax.numpy as jnp
from jax.experimental import pallas as pl


def kernel(x, edge_index, edge_attr, enc_node_w1, enc_node_b1, enc_node_w2, enc_node_b2, enc_edge_w1, enc_edge_b1, enc_edge_w2, enc_edge_b2, proc_edge_w1, proc_edge_b1, proc_edge_w2, proc_edge_b2, proc_node_w1, proc_node_b1, proc_node_w2, proc_node_b2, dec_w1, dec_b1, dec_w2, dec_b2):
    raise NotImplementedError("write your pallas kernel here")


# speedup vs baseline: 1.7555x; 1.7555x over previous
import functools

import jax
import jax.numpy as jnp
from jax import lax
from jax.experimental import pallas as pl
from jax.experimental.pallas import tpu as pltpu
from jax.experimental.pallas import tpu_sc as plsc

N = 50000
E = 800000
H = 128
NUM_LAYERS = 15

NC = 2
NS = 16
NW = NC * NS

GCHUNK = 128
GCPW = 200
E_PAD = NW * GCPW * GCHUNK
GROWS = E_PAD // GCHUNK

R_NODES = 128
NT = 392
N_PAD = NT * R_NODES
SCHUNK = 256
TPW = 13

_f32 = jnp.float32
_i32 = jnp.int32


def _extract(vec, k):
    return vec[k]


def _gather2_body(h_hbm, row_hbm, col_hbm, hr_out, hc_out, idx_all,
                  bufA, bufB, sem, semw):
    wid = lax.axis_index("s") * NC + lax.axis_index("c")
    base = pl.multiple_of(wid * GCPW, 8)

    def do_array(idx_hbm, out_hbm):
        pltpu.sync_copy(idx_hbm.at[pl.ds(base, GCPW)], idx_all)

        def step(s, _):
            def half(buf, g0):
                handles = [
                    pltpu.async_copy(h_hbm.at[idx_all.at[g0 + j]],
                                     buf.at[pl.ds(j * GCHUNK, GCHUNK)], sem)
                    for j in range(2)
                ]
                for hnd in handles:
                    hnd.wait()
                out_row = pl.multiple_of((base + g0) * GCHUNK, 256)
                return pltpu.async_copy(
                    buf, out_hbm.at[pl.ds(out_row, 2 * GCHUNK)], semw)

            wbA = half(bufA, s * 4)
            wbB = half(bufB, s * 4 + 2)
            wbA.wait()
            wbB.wait()
            return 0

        lax.fori_loop(0, GCPW // 4, step, 0)

    do_array(row_hbm, hr_out)
    do_array(col_hbm, hc_out)


@functools.partial(
    pl.kernel,
    out_type=[jax.ShapeDtypeStruct((E_PAD, H), _f32),
              jax.ShapeDtypeStruct((E_PAD, H), _f32)],
    mesh=plsc.VectorSubcoreMesh(core_axis_name="c", subcore_axis_name="s"),
    scratch_types=[
        pltpu.VMEM((GCPW, GCHUNK), _i32),
        pltpu.VMEM((2 * GCHUNK, H), _f32),
        pltpu.VMEM((2 * GCHUNK, H), _f32),
        pltpu.SemaphoreType.DMA,
        pltpu.SemaphoreType.DMA,
    ],
)
def _gather2(h_hbm, row_hbm, col_hbm, hr_out, hc_out, idx_all,
             bufA, bufB, sem, semw):
    _gather2_body(h_hbm, row_hbm, col_hbm, hr_out, hc_out, idx_all,
                  bufA, bufB, sem, semw)


ACC_ROWS = 136


def _segsum_body(hrs_hbm, ens_hbm, colp_hbm, meta_hbm, a1_out, a2_out,
                 meta_v, colv, idxa0, idxa1, idxb0, idxb1, bufA, bufB, zbuf,
                 acc_sh, sem, sem2):
    sid = lax.axis_index("s")
    wid = sid * NC + lax.axis_index("c")
    rowA = pl.multiple_of(sid * 2 * ACC_ROWS, 8)
    rowB = pl.multiple_of(rowA + ACC_ROWS, 8)

    zeros = jnp.zeros((16,), _f32)

    def zrow(n, _):
        for q in range(H // 16):
            zbuf[n, pl.ds(q * 16, 16)] = zeros
        return 0

    lax.fori_loop(0, ACC_ROWS, zrow, 0)

    def task(k, _):
        t = wid + k * NW

        @pl.when(t < NT)
        def _():
            pltpu.sync_copy(meta_hbm.at[pl.ds(pl.multiple_of(t * 16, 16), 16)], meta_v)
            mv = meta_v[...]
            a0 = pl.multiple_of(_extract(mv, 0), SCHUNK)
            nchunks = _extract(mv, 1)
            n0 = pl.multiple_of(_extract(mv, 2), R_NODES)

            pltpu.sync_copy(zbuf, acc_sh.at[pl.ds(rowA, ACC_ROWS)])
            pltpu.sync_copy(zbuf, acc_sh.at[pl.ds(rowB, ACC_ROWS)])

            def chunk(c, _):
                goff = pl.multiple_of(a0 + c * SCHUNK, SCHUNK)
                pltpu.sync_copy(colp_hbm.at[pl.ds(goff, SCHUNK)], colv)
                ca = pltpu.async_copy(hrs_hbm.at[pl.ds(goff, SCHUNK)], bufA, sem)
                cb = pltpu.async_copy(ens_hbm.at[pl.ds(goff, SCHUNK)], bufB, sem)
                for q in range(SCHUNK // 16):
                    cv = colv[pl.ds(q * 16, 16)]
                    d = cv - n0
                    inr = (d >= 0) & (d < R_NODES)
                    loc = jnp.where(inr, d, R_NODES)
                    sl = pl.ds((q % 8) * 16, 16)
                    if q < 8:
                        idxa0[sl] = loc + rowA
                        idxb0[sl] = loc + rowB
                    else:
                        idxa1[sl] = loc + rowA
                        idxb1[sl] = loc + rowB
                ca.wait()
                cb.wait()
                s1 = pltpu.async_copy(bufA.at[pl.ds(0, 128)], acc_sh.at[idxa0],
                                      sem2, add=True)
                s2 = pltpu.async_copy(bufA.at[pl.ds(128, 128)], acc_sh.at[idxa1],
                                      sem2, add=True)
                s3 = pltpu.async_copy(bufB.at[pl.ds(0, 128)], acc_sh.at[idxb0],
                                      sem2, add=True)
                s4 = pltpu.async_copy(bufB.at[pl.ds(128, 128)], acc_sh.at[idxb1],
                                      sem2, add=True)
                s1.wait(); s2.wait(); s3.wait(); s4.wait()
                return 0

            lax.fori_loop(0, nchunks, chunk, 0)

            pltpu.sync_copy(acc_sh.at[pl.ds(rowA, R_NODES)],
                            a1_out.at[pl.ds(n0, R_NODES)])
            pltpu.sync_copy(acc_sh.at[pl.ds(rowB, R_NODES)],
                            a2_out.at[pl.ds(n0, R_NODES)])

        return 0

    lax.fori_loop(0, TPW, task, 0)


@functools.partial(
    pl.kernel,
    out_type=[jax.ShapeDtypeStruct((N_PAD, H), _f32),
              jax.ShapeDtypeStruct((N_PAD, H), _f32)],
    mesh=plsc.VectorSubcoreMesh(core_axis_name="c", subcore_axis_name="s"),
    scratch_types=[
        pltpu.VMEM((16,), _i32),
        pltpu.VMEM((SCHUNK,), _i32),
        pltpu.VMEM((128,), _i32),
        pltpu.VMEM((128,), _i32),
        pltpu.VMEM((128,), _i32),
        pltpu.VMEM((128,), _i32),
        pltpu.VMEM((SCHUNK, H), _f32),
        pltpu.VMEM((SCHUNK, H), _f32),
        pltpu.VMEM((ACC_ROWS, H), _f32),
        pltpu.VMEM_SHARED((NS * 2 * ACC_ROWS, H), _f32),
        pltpu.SemaphoreType.DMA,
        pltpu.SemaphoreType.DMA,
    ],
)
def _segsum(hrs_hbm, ens_hbm, colp_hbm, meta_hbm, a1_out, a2_out,
            meta_v, colv, idxa0, idxa1, idxb0, idxb1, bufA, bufB, zbuf,
            acc_sh, sem, sem2):
    _segsum_body(hrs_hbm, ens_hbm, colp_hbm, meta_hbm, a1_out, a2_out,
                 meta_v, colv, idxa0, idxa1, idxb0, idxb1, bufA, bufB, zbuf,
                 acc_sh, sem, sem2)


def _silu(v):
    return v * jax.nn.sigmoid(v)


def _enc_body(x_ref, w1_ref, b1_ref, w2_ref, b2_ref, o_ref):
    t = jnp.dot(x_ref[...], w1_ref[...], preferred_element_type=_f32)
    t = _silu(t + b1_ref[...])
    o_ref[...] = jnp.dot(t, w2_ref[...], preferred_element_type=_f32) + b2_ref[...]


def _mlp2(x, w1, b1, w2, b2, bm):
    m, fin = x.shape
    fout = b2.shape[-1]
    grid = m // bm
    return pl.pallas_call(
        _enc_body,
        grid=(grid,),
        in_specs=[
            pl.BlockSpec((bm, fin), lambda i: (i, 0)),
            pl.BlockSpec((fin, w1.shape[1]), lambda i: (0, 0)),
            pl.BlockSpec((1, w1.shape[1]), lambda i: (0, 0)),
            pl.BlockSpec((w1.shape[1], fout), lambda i: (0, 0)),
            pl.BlockSpec((1, fout), lambda i: (0, 0)),
        ],
        out_specs=pl.BlockSpec((bm, fout), lambda i: (i, 0)),
        out_shape=jax.ShapeDtypeStruct((m, fout), _f32),
    )(x, w1, b1.reshape(1, -1), w2, b2.reshape(1, -1))


def _edge_mlp_body(hr_ref, hc_ref, e_ref, wa, wb, wc, b1, w2, b2, o_ref):
    acc = jnp.dot(hr_ref[...], wa[...], preferred_element_type=_f32)
    acc += jnp.dot(hc_ref[...], wb[...], preferred_element_type=_f32)
    acc += jnp.dot(e_ref[...], wc[...], preferred_element_type=_f32)
    t = _silu(acc + b1[...])
    o_ref[...] = (jnp.dot(t, w2[...], preferred_element_type=_f32)
                  + b2[...] + e_ref[...])


def _edge_mlp(hr, hc, e, wa, wb, wc, b1, w2, b2):
    bm = 2048
    grid = E_PAD // bm
    wspec = pl.BlockSpec((H, H), lambda i: (0, 0))
    bspec = pl.BlockSpec((1, H), lambda i: (0, 0))
    dspec = pl.BlockSpec((bm, H), lambda i: (i, 0))
    return pl.pallas_call(
        _edge_mlp_body,
        grid=(grid,),
        in_specs=[dspec, dspec, dspec, wspec, wspec, wspec, bspec, wspec, bspec],
        out_specs=dspec,
        out_shape=jax.ShapeDtypeStruct((E_PAD, H), _f32),
    )(hr, hc, e, wa, wb, wc, b1.reshape(1, -1), w2, b2.reshape(1, -1))


def _node_mlp_body(h_ref, a1_ref, a2_ref, wa, wb, wc, b1, w2, b2, o_ref):
    acc = jnp.dot(h_ref[...], wa[...], preferred_element_type=_f32)
    acc += jnp.dot(a1_ref[...], wb[...], preferred_element_type=_f32)
    acc += jnp.dot(a2_ref[...], wc[...], preferred_element_type=_f32)
    t = _silu(acc + b1[...])
    o_ref[...] = (jnp.dot(t, w2[...], preferred_element_type=_f32)
                  + b2[...] + h_ref[...])


def _node_mlp(h, a1, a2, wa, wb, wc, b1, w2, b2):
    bm = 2000
    grid = N // bm
    wspec = pl.BlockSpec((H, H), lambda i: (0, 0))
    bspec = pl.BlockSpec((1, H), lambda i: (0, 0))
    dspec = pl.BlockSpec((bm, H), lambda i: (i, 0))
    return pl.pallas_call(
        _node_mlp_body,
        grid=(grid,),
        in_specs=[dspec, dspec, dspec, wspec, wspec, wspec, bspec, wspec, bspec],
        out_specs=dspec,
        out_shape=jax.ShapeDtypeStruct((N, H), _f32),
    )(h, a1, a2, wa, wb, wc, b1.reshape(1, -1), w2, b2.reshape(1, -1))


def kernel(x, edge_index, edge_attr,
           enc_node_w1, enc_node_b1, enc_node_w2, enc_node_b2,
           enc_edge_w1, enc_edge_b1, enc_edge_w2, enc_edge_b2,
           proc_edge_w1, proc_edge_b1, proc_edge_w2, proc_edge_b2,
           proc_node_w1, proc_node_b1, proc_node_w2, proc_node_b2,
           dec_w1, dec_b1, dec_w2, dec_b2):
    row = edge_index[0]
    col = edge_index[1]

    perm = jnp.argsort(col)
    col_s = col[perm]
    row_s = row[perm]
    ea_s = edge_attr[perm]

    row_p = jnp.pad(row_s, (0, E_PAD - E)).reshape(GROWS, GCHUNK)
    col_p = jnp.pad(col_s, (0, E_PAD - E)).reshape(GROWS, GCHUNK)
    colp = jnp.pad(col_s, (0, E_PAD - E), constant_values=N_PAD)

    bounds = (jnp.arange(NT + 1, dtype=_i32) * R_NODES).clip(0, N)
    ptr = jnp.searchsorted(col_s, bounds, side="left").astype(_i32)
    p0, p1 = ptr[:-1], ptr[1:]
    a0 = (p0 // SCHUNK) * SCHUNK
    nchunks = (p1 - a0 + SCHUNK - 1) // SCHUNK
    n0 = jnp.arange(NT, dtype=_i32) * R_NODES
    meta = jnp.stack([a0, nchunks, n0] + [jnp.zeros(NT, _i32)] * 13,
                     axis=1).reshape(-1)

    pe_a, pe_b, pe_c = (proc_edge_w1[:H], proc_edge_w1[H:2 * H],
                        proc_edge_w1[2 * H:])
    pn_a, pn_b, pn_c = (proc_node_w1[:H], proc_node_w1[H:2 * H],
                        proc_node_w1[2 * H:])

    xp = jnp.pad(x, ((0, 0), (0, 3)))
    enw1 = jnp.pad(enc_node_w1, ((0, 3), (0, 0)))
    h = _mlp2(xp, enw1, enc_node_b1, enc_node_w2, enc_node_b2, bm=2000)

    eap = jnp.pad(ea_s, ((0, E_PAD - E), (0, 5)))
    eew1 = jnp.pad(enc_edge_w1, ((0, 5), (0, 0)))
    e = _mlp2(eap, eew1, enc_edge_b1, enc_edge_w2, enc_edge_b2, bm=2048)

    for _ in range(NUM_LAYERS):
        hr, hc = _gather2(h, row_p, col_p)
        e = _edge_mlp(hr, hc, e, pe_a, pe_b, pe_c, proc_edge_b1,
                      proc_edge_w2, proc_edge_b2)
        a1, a2 = _segsum(hr, e, colp, meta)
        h = _node_mlp(h, a1, a2, pn_a, pn_b, pn_c, proc_node_b1,
                      proc_node_w2, proc_node_b2)

    return _mlp2(h, dec_w1, dec_b1, dec_w2, dec_b2, bm=2000)


# speedup vs baseline: 1.8022x; 1.0266x over previous
import functools

import jax
import jax.numpy as jnp
from jax import lax
from jax.experimental import pallas as pl
from jax.experimental.pallas import tpu as pltpu
from jax.experimental.pallas import tpu_sc as plsc

N = 50000
E = 800000
H = 128
NUM_LAYERS = 15

NC = 2
NS = 16
NW = NC * NS

GCHUNK = 128
GCPW = 200
E_PAD = NW * GCPW * GCHUNK
GROWS = E_PAD // GCHUNK

R_NODES = 128
NT = 392
N_PAD = NT * R_NODES
SCHUNK = 256
TPW = 13
ACC_ROWS = 136

_f32 = jnp.float32
_i32 = jnp.int32


def _gather2_body(h_hbm, row_hbm, col_hbm, hr_out, hc_out, idx_all,
                  bufA, bufB, sem, semw):
    wid = lax.axis_index("s") * NC + lax.axis_index("c")
    base = pl.multiple_of(wid * GCPW, 8)

    def do_array(idx_hbm, out_hbm):
        pltpu.sync_copy(idx_hbm.at[pl.ds(base, GCPW)], idx_all)

        def half(buf, g0, width):
            handles = [
                pltpu.async_copy(h_hbm.at[idx_all.at[g0 + j]],
                                 buf.at[pl.ds(j * GCHUNK, GCHUNK)], sem)
                for j in range(width)
            ]
            for hnd in handles:
                hnd.wait()
            out_row = pl.multiple_of((base + g0) * GCHUNK, GCHUNK)
            return pltpu.async_copy(
                buf.at[pl.ds(0, width * GCHUNK)],
                out_hbm.at[pl.ds(out_row, width * GCHUNK)], semw)

        def step(s, _):
            wbA = half(bufA, s * 6, 3)
            wbB = half(bufB, s * 6 + 3, 3)
            wbA.wait()
            wbB.wait()
            return 0

        lax.fori_loop(0, GCPW // 6, step, 0)
        wbA = half(bufA, 198, 2)
        wbA.wait()

    do_array(row_hbm, hr_out)
    do_array(col_hbm, hc_out)


@functools.partial(
    pl.kernel,
    out_type=[jax.ShapeDtypeStruct((E_PAD, H), _f32),
              jax.ShapeDtypeStruct((E_PAD, H), _f32)],
    mesh=plsc.VectorSubcoreMesh(core_axis_name="c", subcore_axis_name="s"),
    scratch_types=[
        pltpu.VMEM((GCPW, GCHUNK), _i32),
        pltpu.VMEM((3 * GCHUNK, H), _f32),
        pltpu.VMEM((3 * GCHUNK, H), _f32),
        pltpu.SemaphoreType.DMA,
        pltpu.SemaphoreType.DMA,
    ],
)
def _gather2(h_hbm, row_hbm, col_hbm, hr_out, hc_out, idx_all,
             bufA, bufB, sem, semw):
    _gather2_body(h_hbm, row_hbm, col_hbm, hr_out, hc_out, idx_all,
                  bufA, bufB, sem, semw)


def _segsum_body(hrs_hbm, ens_hbm, colp_hbm, meta_hbm, a1_out, a2_out,
                 meta_v, colv, idxa0, idxa1, idxb0, idxb1, bufA, bufB, zbuf,
                 acc_sh, sem, sem2):
    sid = lax.axis_index("s")
    wid = sid * NC + lax.axis_index("c")
    rowA = pl.multiple_of(sid * 2 * ACC_ROWS, 8)
    rowB = pl.multiple_of(rowA + ACC_ROWS, 8)

    zeros = jnp.zeros((16,), _f32)

    def zrow(n, _):
        for q in range(H // 16):
            zbuf[n, pl.ds(q * 16, 16)] = zeros
        return 0

    lax.fori_loop(0, ACC_ROWS, zrow, 0)

    def task(k, _):
        t = wid + k * NW

        @pl.when(t < NT)
        def _():
            pltpu.sync_copy(meta_hbm.at[pl.ds(pl.multiple_of(t * 16, 16), 16)],
                            meta_v)
            mv = meta_v[...]
            a0 = pl.multiple_of(mv[0], SCHUNK)
            nchunks = mv[1]
            n0 = pl.multiple_of(mv[2], R_NODES)

            pltpu.sync_copy(zbuf, acc_sh.at[pl.ds(rowA, ACC_ROWS)])
            pltpu.sync_copy(zbuf, acc_sh.at[pl.ds(rowB, ACC_ROWS)])

            def chunk(c, _):
                goff = pl.multiple_of(a0 + c * SCHUNK, SCHUNK)
                ca = pltpu.async_copy(hrs_hbm.at[pl.ds(goff, SCHUNK)], bufA, sem)
                cb = pltpu.async_copy(ens_hbm.at[pl.ds(goff, SCHUNK)], bufB, sem)
                pltpu.sync_copy(colp_hbm.at[pl.ds(goff, SCHUNK)], colv)
                for q in range(SCHUNK // 16):
                    cv = colv[pl.ds(q * 16, 16)]
                    d = cv - n0
                    inr = (d >= 0) & (d < R_NODES)
                    loc = jnp.where(inr, d, R_NODES)
                    sl = pl.ds((q % 8) * 16, 16)
                    if q < 8:
                        idxa0[sl] = loc + rowA
                        idxb0[sl] = loc + rowB
                    else:
                        idxa1[sl] = loc + rowA
                        idxb1[sl] = loc + rowB
                ca.wait()
                cb.wait()
                s1 = pltpu.async_copy(bufA.at[pl.ds(0, 128)], acc_sh.at[idxa0],
                                      sem2, add=True)
                s2 = pltpu.async_copy(bufA.at[pl.ds(128, 128)], acc_sh.at[idxa1],
                                      sem2, add=True)
                s3 = pltpu.async_copy(bufB.at[pl.ds(0, 128)], acc_sh.at[idxb0],
                                      sem2, add=True)
                s4 = pltpu.async_copy(bufB.at[pl.ds(128, 128)], acc_sh.at[idxb1],
                                      sem2, add=True)
                s1.wait(); s2.wait(); s3.wait(); s4.wait()
                return 0

            lax.fori_loop(0, nchunks, chunk, 0)

            pltpu.sync_copy(acc_sh.at[pl.ds(rowA, R_NODES)],
                            a1_out.at[pl.ds(n0, R_NODES)])
            pltpu.sync_copy(acc_sh.at[pl.ds(rowB, R_NODES)],
                            a2_out.at[pl.ds(n0, R_NODES)])

        return 0

    lax.fori_loop(0, TPW, task, 0)


@functools.partial(
    pl.kernel,
    out_type=[jax.ShapeDtypeStruct((N_PAD, H), _f32),
              jax.ShapeDtypeStruct((N_PAD, H), _f32)],
    mesh=plsc.VectorSubcoreMesh(core_axis_name="c", subcore_axis_name="s"),
    scratch_types=[
        pltpu.VMEM((16,), _i32),
        pltpu.VMEM((SCHUNK,), _i32),
        pltpu.VMEM((128,), _i32),
        pltpu.VMEM((128,), _i32),
        pltpu.VMEM((128,), _i32),
        pltpu.VMEM((128,), _i32),
        pltpu.VMEM((SCHUNK, H), _f32),
        pltpu.VMEM((SCHUNK, H), _f32),
        pltpu.VMEM((ACC_ROWS, H), _f32),
        pltpu.VMEM_SHARED((NS * 2 * ACC_ROWS, H), _f32),
        pltpu.SemaphoreType.DMA,
        pltpu.SemaphoreType.DMA,
    ],
)
def _segsum(hrs_hbm, ens_hbm, colp_hbm, meta_hbm, a1_out, a2_out,
            meta_v, colv, idxa0, idxa1, idxb0, idxb1, bufA, bufB, zbuf,
            acc_sh, sem, sem2):
    _segsum_body(hrs_hbm, ens_hbm, colp_hbm, meta_hbm, a1_out, a2_out,
                 meta_v, colv, idxa0, idxa1, idxb0, idxb1, bufA, bufB, zbuf,
                 acc_sh, sem, sem2)


def _silu(v):
    return v * jax.nn.sigmoid(v)


def _enc_body(x_ref, w1_ref, b1_ref, w2_ref, b2_ref, o_ref):
    t = jnp.dot(x_ref[...], w1_ref[...], preferred_element_type=_f32)
    t = _silu(t + b1_ref[...])
    o_ref[...] = jnp.dot(t, w2_ref[...], preferred_element_type=_f32) + b2_ref[...]


def _mlp2(x, w1, b1, w2, b2, bm):
    m, fin = x.shape
    fout = b2.shape[-1]
    grid = m // bm
    return pl.pallas_call(
        _enc_body,
        grid=(grid,),
        in_specs=[
            pl.BlockSpec((bm, fin), lambda i: (i, 0)),
            pl.BlockSpec((fin, w1.shape[1]), lambda i: (0, 0)),
            pl.BlockSpec((1, w1.shape[1]), lambda i: (0, 0)),
            pl.BlockSpec((w1.shape[1], fout), lambda i: (0, 0)),
            pl.BlockSpec((1, fout), lambda i: (0, 0)),
        ],
        out_specs=pl.BlockSpec((bm, fout), lambda i: (i, 0)),
        out_shape=jax.ShapeDtypeStruct((m, fout), _f32),
    )(x, w1, b1.reshape(1, -1), w2, b2.reshape(1, -1))


def _edge_mlp_body(hr_ref, hc_ref, e_ref, wa, wb, wc, b1, w2, b2, o_ref):
    acc = jnp.dot(hr_ref[...], wa[...], preferred_element_type=_f32)
    acc += jnp.dot(hc_ref[...], wb[...], preferred_element_type=_f32)
    acc += jnp.dot(e_ref[...], wc[...], preferred_element_type=_f32)
    t = _silu(acc + b1[...])
    o_ref[...] = (jnp.dot(t, w2[...], preferred_element_type=_f32)
                  + b2[...] + e_ref[...])


def _edge_mlp(hr, hc, e, wa, wb, wc, b1, w2, b2):
    bm = 2048
    grid = E_PAD // bm
    wspec = pl.BlockSpec((H, H), lambda i: (0, 0))
    bspec = pl.BlockSpec((1, H), lambda i: (0, 0))
    dspec = pl.BlockSpec((bm, H), lambda i: (i, 0))
    return pl.pallas_call(
        _edge_mlp_body,
        grid=(grid,),
        in_specs=[dspec, dspec, dspec, wspec, wspec, wspec, bspec, wspec, bspec],
        out_specs=dspec,
        out_shape=jax.ShapeDtypeStruct((E_PAD, H), _f32),
    )(hr, hc, e, wa, wb, wc, b1.reshape(1, -1), w2, b2.reshape(1, -1))


def _node_mlp_body(h_ref, a1_ref, a2_ref, wa, wb, wc, b1, w2, b2, o_ref):
    acc = jnp.dot(h_ref[...], wa[...], preferred_element_type=_f32)
    acc += jnp.dot(a1_ref[...], wb[...], preferred_element_type=_f32)
    acc += jnp.dot(a2_ref[...], wc[...], preferred_element_type=_f32)
    t = _silu(acc + b1[...])
    o_ref[...] = (jnp.dot(t, w2[...], preferred_element_type=_f32)
                  + b2[...] + h_ref[...])


def _node_mlp(h, a1, a2, wa, wb, wc, b1, w2, b2):
    bm = 2000
    grid = N // bm
    wspec = pl.BlockSpec((H, H), lambda i: (0, 0))
    bspec = pl.BlockSpec((1, H), lambda i: (0, 0))
    dspec = pl.BlockSpec((bm, H), lambda i: (i, 0))
    return pl.pallas_call(
        _node_mlp_body,
        grid=(grid,),
        in_specs=[dspec, dspec, dspec, wspec, wspec, wspec, bspec, wspec, bspec],
        out_specs=dspec,
        out_shape=jax.ShapeDtypeStruct((N, H), _f32),
    )(h, a1, a2, wa, wb, wc, b1.reshape(1, -1), w2, b2.reshape(1, -1))


def kernel(x, edge_index, edge_attr,
           enc_node_w1, enc_node_b1, enc_node_w2, enc_node_b2,
           enc_edge_w1, enc_edge_b1, enc_edge_w2, enc_edge_b2,
           proc_edge_w1, proc_edge_b1, proc_edge_w2, proc_edge_b2,
           proc_node_w1, proc_node_b1, proc_node_w2, proc_node_b2,
           dec_w1, dec_b1, dec_w2, dec_b2):
    row = edge_index[0]
    col = edge_index[1]

    perm = jnp.argsort(col)
    col_s = col[perm]
    row_s = row[perm]
    ea_s = edge_attr[perm]

    row_p = jnp.pad(row_s, (0, E_PAD - E)).reshape(GROWS, GCHUNK)
    col_p = jnp.pad(col_s, (0, E_PAD - E)).reshape(GROWS, GCHUNK)
    colp = jnp.pad(col_s, (0, E_PAD - E), constant_values=N_PAD)

    bounds = (jnp.arange(NT + 1, dtype=_i32) * R_NODES).clip(0, N)
    ptr = jnp.searchsorted(col_s, bounds, side="left").astype(_i32)
    p0, p1 = ptr[:-1], ptr[1:]
    a0 = (p0 // SCHUNK) * SCHUNK
    nchunks = (p1 - a0 + SCHUNK - 1) // SCHUNK
    n0 = jnp.arange(NT, dtype=_i32) * R_NODES
    meta = jnp.stack([a0, nchunks, n0] + [jnp.zeros(NT, _i32)] * 13,
                     axis=1).reshape(-1)

    pe_a, pe_b, pe_c = (proc_edge_w1[:H], proc_edge_w1[H:2 * H],
                        proc_edge_w1[2 * H:])
    pn_a, pn_b, pn_c = (proc_node_w1[:H], proc_node_w1[H:2 * H],
                        proc_node_w1[2 * H:])

    xp = jnp.pad(x, ((0, 0), (0, 3)))
    enw1 = jnp.pad(enc_node_w1, ((0, 3), (0, 0)))
    h = _mlp2(xp, enw1, enc_node_b1, enc_node_w2, enc_node_b2, bm=2000)

    eap = jnp.pad(ea_s, ((0, E_PAD - E), (0, 5)))
    eew1 = jnp.pad(enc_edge_w1, ((0, 5), (0, 0)))
    e = _mlp2(eap, eew1, enc_edge_b1, enc_edge_w2, enc_edge_b2, bm=2048)

    for _ in range(NUM_LAYERS):
        hr, hc = _gather2(h, row_p, col_p)
        e = _edge_mlp(hr, hc, e, pe_a, pe_b, pe_c, proc_edge_b1,
                      proc_edge_w2, proc_edge_b2)
        a1, a2 = _segsum(hr, e, colp, meta)
        h = _node_mlp(h, a1, a2, pn_a, pn_b, pn_c, proc_node_b1,
                      proc_node_w2, proc_node_b2)

    return _mlp2(h, dec_w1, dec_b1, dec_w2, dec_b2, bm=2000)
